# baseline (speedup 1.0000x reference)
import functools

import jax
import jax.numpy as jnp
from jax import lax
from jax.experimental import pallas as pl
from jax.experimental.pallas import tpu as pltpu
from jax.experimental.pallas import tpu_sc as plsc

TM = 1024
CB = 8192
_LANE = 128
_INT_MAX = jnp.iinfo(jnp.int32).max

_NC = 2
_NS = 16
_NW = _NC * _NS
_IDX_CHUNK = 128


def _argmin_body(
    x2_ref, flat_ref, embed_ref, e2_ref, colf_ref, codes_ref, val_ref, idx_ref
):
    c = pl.program_id(1)
    ncb = pl.num_programs(1)

    @pl.when(c == 0)
    def _init():
        val_ref[...] = jnp.full((TM, 1), jnp.inf, jnp.float32)
        idx_ref[...] = jnp.zeros((TM, 1), jnp.float32)

    dot2 = lax.dot_general(
        flat_ref[...] * -2.0, embed_ref[...], (((1,), (1,)), ((), ()))
    )
    x2e2 = x2_ref[...] + e2_ref[...]
    d = x2e2 + dot2
    bmin = jnp.min(d, axis=1, keepdims=True)
    bidx = jnp.min(
        jnp.where(d == bmin, colf_ref[...], jnp.inf), axis=1, keepdims=True
    )
    better = bmin < val_ref[...]
    val_ref[...] = jnp.where(better, bmin, val_ref[...])
    idx_ref[...] = jnp.where(better, bidx, idx_ref[...])

    @pl.when(c == ncb - 1)
    def _emit():
        codes_ref[...] = idx_ref[...].astype(jnp.int32)


def _codes_call(x2, flat, embed, e2, colf):
    n, c = flat.shape
    v = embed.shape[0]
    return pl.pallas_call(
        _argmin_body,
        grid=(n // TM, v // CB),
        in_specs=[
            pl.BlockSpec((TM, 1), lambda t, cb: (t, 0)),
            pl.BlockSpec((TM, c), lambda t, cb: (t, 0)),
            pl.BlockSpec((CB, c), lambda t, cb: (cb, 0)),
            pl.BlockSpec((1, CB), lambda t, cb: (0, cb)),
            pl.BlockSpec((1, CB), lambda t, cb: (0, cb)),
        ],
        out_specs=pl.BlockSpec((TM, 1), lambda t, cb: (t, 0)),
        out_shape=jax.ShapeDtypeStruct((n, 1), jnp.int32),
        scratch_shapes=[
            pltpu.VMEM((TM, 1), jnp.float32),
            pltpu.VMEM((TM, 1), jnp.float32),
        ],
        compiler_params=pltpu.CompilerParams(
            dimension_semantics=("arbitrary", "arbitrary")
        ),
    )(x2, flat, embed, e2, colf)


def _make_sc_gather(v, d, b):
    b_per_w = b // _NW
    chunks = b_per_w // _IDX_CHUNK
    mesh = plsc.VectorSubcoreMesh(core_axis_name="c", subcore_axis_name="s")

    @functools.partial(
        pl.kernel,
        mesh=mesh,
        out_type=jax.ShapeDtypeStruct((_NW, chunks, _IDX_CHUNK, d), jnp.float32),
        scratch_types=[
            pltpu.VMEM((chunks, _IDX_CHUNK), jnp.int32),
            pltpu.VMEM((chunks, _IDX_CHUNK, d), jnp.float32),
            pltpu.SemaphoreType.DMA,
        ],
    )
    def gk(table_hbm, idx_hbm, out_hbm, idx_v, rows_v, sem):
        wid = lax.axis_index("s") * _NC + lax.axis_index("c")
        pltpu.sync_copy(idx_hbm.at[pl.ds(wid * chunks, chunks)], idx_v)
        copies = []
        for j in range(chunks):
            copies.append(
                pltpu.async_copy(table_hbm.at[idx_v.at[j]], rows_v.at[j], sem)
            )
        for cp in copies:
            cp.wait()
        pltpu.sync_copy(rows_v, out_hbm.at[wid])

    return gk


def kernel(z, embed):
    b, c, h, w = z.shape
    v = embed.shape[0]
    flat = jnp.transpose(z, (0, 2, 3, 1)).reshape(-1, c)
    n = flat.shape[0]
    x2 = jnp.sum(flat * flat, axis=1, keepdims=True)
    e2 = jnp.sum(embed * embed, axis=1, keepdims=True).T
    colf = jnp.arange(v, dtype=jnp.float32).reshape(1, v)

    nh = n // 2
    bh = b // 2
    gather = _make_sc_gather(v, c, nh)
    codes_halves, zq_halves = [], []
    for i in range(2):
        sl = slice(i * nh, (i + 1) * nh)
        cds = _codes_call(x2[sl], flat[sl], embed, e2, colf).reshape(nh)
        zq_rows = gather(embed, cds.reshape(nh // _IDX_CHUNK, _IDX_CHUNK))
        codes_halves.append(cds)
        zq_halves.append(
            jnp.transpose(zq_rows.reshape(bh, h, w, c), (0, 3, 1, 2))
        )
    codes = jnp.concatenate(codes_halves)
    z_q = jnp.concatenate(zq_halves, axis=0)
    return (z_q, codes.reshape(b, h, w))


# speedup vs baseline: 1.1561x; 1.1561x over previous
import functools

import jax
import jax.numpy as jnp
from jax import lax
from jax.experimental import pallas as pl
from jax.experimental.pallas import tpu as pltpu
from jax.experimental.pallas import tpu_sc as plsc

TM = 1024
CB = 8192
_LANE = 128
_INT_MAX = jnp.iinfo(jnp.int32).max

_NC = 2
_NS = 16
_NW = _NC * _NS
_IDX_CHUNK = 128


def _argmin_body(
    x2_ref, zb_ref, embed_ref, e2_ref, colf_ref, codes_ref, val_ref, idx_ref
):
    c = pl.program_id(1)
    ncb = pl.num_programs(1)

    @pl.when(c == 0)
    def _init():
        val_ref[...] = jnp.full((TM, 1), jnp.inf, jnp.float32)
        idx_ref[...] = jnp.zeros((TM, 1), jnp.float32)

    dot2 = lax.dot_general(
        zb_ref[0] * -2.0, embed_ref[...], (((0,), (1,)), ((), ()))
    )
    x2e2 = x2_ref[...] + e2_ref[...]
    d = x2e2 + dot2
    bmin = jnp.min(d, axis=1, keepdims=True)
    bidx = jnp.min(
        jnp.where(d == bmin, colf_ref[...], jnp.inf), axis=1, keepdims=True
    )
    better = bmin < val_ref[...]
    val_ref[...] = jnp.where(better, bmin, val_ref[...])
    idx_ref[...] = jnp.where(better, bidx, idx_ref[...])

    @pl.when(c == ncb - 1)
    def _emit():
        codes_ref[...] = idx_ref[...].astype(jnp.int32)


def _codes_call(x2, z3, embed, e2, colf):
    b, c, hw = z3.shape
    n = b * hw
    v = embed.shape[0]
    assert hw == TM
    return pl.pallas_call(
        _argmin_body,
        grid=(n // TM, v // CB),
        in_specs=[
            pl.BlockSpec((TM, 1), lambda t, cb: (t, 0)),
            pl.BlockSpec((1, c, TM), lambda t, cb: (t, 0, 0)),
            pl.BlockSpec((CB, c), lambda t, cb: (cb, 0)),
            pl.BlockSpec((1, CB), lambda t, cb: (0, cb)),
            pl.BlockSpec((1, CB), lambda t, cb: (0, cb)),
        ],
        out_specs=pl.BlockSpec((TM, 1), lambda t, cb: (t, 0)),
        out_shape=jax.ShapeDtypeStruct((n, 1), jnp.int32),
        scratch_shapes=[
            pltpu.VMEM((TM, 1), jnp.float32),
            pltpu.VMEM((TM, 1), jnp.float32),
        ],
        compiler_params=pltpu.CompilerParams(
            dimension_semantics=("arbitrary", "arbitrary")
        ),
    )(x2, z3, embed, e2, colf)


def _make_sc_gather(v, d, b):
    b_per_w = b // _NW
    chunks = b_per_w // _IDX_CHUNK
    mesh = plsc.VectorSubcoreMesh(core_axis_name="c", subcore_axis_name="s")

    @functools.partial(
        pl.kernel,
        mesh=mesh,
        out_type=jax.ShapeDtypeStruct((_NW, chunks, _IDX_CHUNK, d), jnp.float32),
        scratch_types=[
            pltpu.VMEM((chunks, _IDX_CHUNK), jnp.int32),
            pltpu.VMEM((chunks, _IDX_CHUNK, d), jnp.float32),
            pltpu.SemaphoreType.DMA,
        ],
    )
    def gk(table_hbm, idx_hbm, out_hbm, idx_v, rows_v, sem):
        wid = lax.axis_index("s") * _NC + lax.axis_index("c")
        pltpu.sync_copy(idx_hbm.at[pl.ds(wid * chunks, chunks)], idx_v)
        copies = []
        for j in range(chunks):
            copies.append(
                pltpu.async_copy(table_hbm.at[idx_v.at[j]], rows_v.at[j], sem)
            )
        for cp in copies:
            cp.wait()
        pltpu.sync_copy(rows_v, out_hbm.at[wid])

    return gk


def kernel(z, embed):
    b, c, h, w = z.shape
    v = embed.shape[0]
    flat = jnp.transpose(z, (0, 2, 3, 1)).reshape(-1, c)
    n = flat.shape[0]
    x2 = jnp.sum(flat * flat, axis=1, keepdims=True)
    e2 = jnp.sum(embed * embed, axis=1, keepdims=True).T
    colf = jnp.arange(v, dtype=jnp.float32).reshape(1, v)

    codes = _codes_call(x2, z.reshape(b, c, h * w), embed, e2, colf).reshape(n)

    idx2d = codes.reshape(n // _IDX_CHUNK, _IDX_CHUNK)
    zq_rows = _make_sc_gather(v, c, n)(embed, idx2d)
    z_q = jnp.transpose(zq_rows.reshape(b, h, w, c), (0, 3, 1, 2))
    return (z_q, codes.reshape(b, h, w))


# speedup vs baseline: 1.2222x; 1.0572x over previous
import functools

import jax
import jax.numpy as jnp
from jax import lax
from jax.experimental import pallas as pl
from jax.experimental.pallas import tpu as pltpu
from jax.experimental.pallas import tpu_sc as plsc

TM = 1024
CB = 8192
_LANE = 128
_INT_MAX = jnp.iinfo(jnp.int32).max

_NC = 2
_NS = 16
_NW = _NC * _NS
_IDX_CHUNK = 128


def _argmin_body(x2_ref, flat_ref, embed_ref, e2_ref, colf_ref, codes_ref):
    dot2 = lax.dot_general(
        flat_ref[...] * -2.0, embed_ref[...], (((1,), (1,)), ((), ()))
    )
    d = (x2_ref[...] + e2_ref[...]) + dot2
    bmin = jnp.min(d, axis=1, keepdims=True)
    bidx = jnp.min(
        jnp.where(d == bmin, colf_ref[...], jnp.inf), axis=1, keepdims=True
    )
    codes_ref[...] = bidx.astype(jnp.int32)


def _codes_call(x2, flat, embed, e2, colf):
    n, c = flat.shape
    v = embed.shape[0]
    assert v == CB
    return pl.pallas_call(
        _argmin_body,
        grid=(n // TM,),
        in_specs=[
            pl.BlockSpec((TM, 1), lambda t: (t, 0)),
            pl.BlockSpec((TM, c), lambda t: (t, 0)),
            pl.BlockSpec((CB, c), lambda t: (0, 0)),
            pl.BlockSpec((1, CB), lambda t: (0, 0)),
            pl.BlockSpec((1, CB), lambda t: (0, 0)),
        ],
        out_specs=pl.BlockSpec((TM, 1), lambda t: (t, 0)),
        out_shape=jax.ShapeDtypeStruct((n, 1), jnp.int32),
        compiler_params=pltpu.CompilerParams(
            dimension_semantics=("arbitrary",)
        ),
    )(x2, flat, embed, e2, colf)


def _make_sc_gather(v, d, b):
    b_per_w = b // _NW
    chunks = b_per_w // _IDX_CHUNK
    mesh = plsc.VectorSubcoreMesh(core_axis_name="c", subcore_axis_name="s")

    @functools.partial(
        pl.kernel,
        mesh=mesh,
        out_type=jax.ShapeDtypeStruct((_NW, chunks, _IDX_CHUNK, d), jnp.float32),
        scratch_types=[
            pltpu.VMEM((chunks, _IDX_CHUNK), jnp.int32),
            pltpu.VMEM((chunks, _IDX_CHUNK, d), jnp.float32),
            pltpu.SemaphoreType.DMA,
        ],
    )
    def gk(table_hbm, idx_hbm, out_hbm, idx_v, rows_v, sem):
        wid = lax.axis_index("s") * _NC + lax.axis_index("c")
        pltpu.sync_copy(idx_hbm.at[pl.ds(wid * chunks, chunks)], idx_v)
        copies = []
        for j in range(chunks):
            copies.append(
                pltpu.async_copy(table_hbm.at[idx_v.at[j]], rows_v.at[j], sem)
            )
        for cp in copies:
            cp.wait()
        pltpu.sync_copy(rows_v, out_hbm.at[wid])

    return gk


def kernel(z, embed):
    b, c, h, w = z.shape
    v = embed.shape[0]
    flat = jnp.transpose(z, (0, 2, 3, 1)).reshape(-1, c)
    n = flat.shape[0]
    x2 = jnp.sum(flat * flat, axis=1, keepdims=True)
    e2 = jnp.sum(embed * embed, axis=1, keepdims=True).T
    colf = jnp.arange(v, dtype=jnp.float32).reshape(1, v)

    codes = _codes_call(x2, flat, embed, e2, colf).reshape(n)

    idx2d = codes.reshape(n // _IDX_CHUNK, _IDX_CHUNK)
    zq_rows = _make_sc_gather(v, c, n)(embed, idx2d)
    z_q = jnp.transpose(zq_rows.reshape(b, h, w, c), (0, 3, 1, 2))
    return (z_q, codes.reshape(b, h, w))
